# baseline (speedup 1.0000x reference)
import jax
import jax.numpy as jnp
from jax.experimental import pallas as pl
from jax.experimental.pallas import tpu as pltpu

_OFFSETS = tuple(list(range(33)) + [48, 64, 96, 128, 192, 256, 384, 512, 768, 1024, 1536])
_LB = 128
_NCOV = 37
_STRIDED = tuple(o for o in _OFFSETS if o > _LB)
_NEG = -1e30
_TILE = 128
_KWIN = _TILE + _LB
_N = 2048
_SC = 0.125


def _tree_sum(terms):
    while len(terms) > 1:
        nxt = [a + b for a, b in zip(terms[::2], terms[1::2])]
        if len(terms) % 2:
            nxt.append(terms[-1])
        terms = nxt
    return terms[0]


def _tree_max(terms):
    while len(terms) > 1:
        nxt = [jnp.maximum(a, b) for a, b in zip(terms[::2], terms[1::2])]
        if len(terms) % 2:
            nxt.append(terms[-1])
        terms = nxt
    return terms[0]


def _attn_body(ws_ref, w_ref, q4_ref, k4_ref, v4_ref, out4_ref):
    q_ref = q4_ref.at[0]
    k_ref = k4_ref.at[0]
    v_ref = v4_ref.at[0]
    out_ref = out4_ref.at[0]
    wband = w_ref[0]
    wpos = wband > 0.0
    ones_den = jnp.ones((_KWIN, 8), jnp.bfloat16)

    c2 = jax.lax.broadcasted_iota(jnp.int32, (_TILE, _KWIN), 1)
    rcol = jax.lax.broadcasted_iota(jnp.int32, (_TILE, 1), 0)

    for t in range(_N // _TILE):
        n0 = t * _TILE
        qb = q_ref[0, pl.ds(n0, _TILE), :] * _SC
        if t == 0:
            kw = jnp.concatenate(
                [k_ref[0, 0:_TILE, :], k_ref[0, 0:_TILE, :]], axis=0)
            vw = jnp.concatenate(
                [v_ref[0, 0:_TILE, :], v_ref[0, 0:_TILE, :]], axis=0)
        else:
            kw = k_ref[0, pl.ds(n0 - _LB, _KWIN), :]
            vw = v_ref[0, pl.ds(n0 - _LB, _KWIN), :]
        s_win = jax.lax.dot_general(
            qb, kw, (((1,), (1,)), ((), ())),
            preferred_element_type=jnp.float32)
        if t == 0:
            valid = wpos & (c2 >= _LB)
        else:
            valid = wpos
        s_win = jnp.where(valid, s_win, _NEG)
        m_win = jnp.max(s_win, axis=1, keepdims=True)

        live = [(j, off) for j, off in enumerate(_STRIDED) if n0 + _TILE > off]
        s_str = {}
        for j, off in live:
            if n0 >= off:
                kb = k_ref[0, pl.ds(n0 - off, _TILE), :]
            else:
                d = off - n0
                kb = jnp.concatenate(
                    [k_ref[0, 0:d, :], k_ref[0, 0:_TILE - d, :]], axis=0)
            sj = jnp.sum(qb * kb, axis=1, keepdims=True)
            if n0 < off:
                sj = jnp.where(rcol >= off - n0, sj, _NEG)
            s_str[j] = sj
        m = _tree_max([m_win] + list(s_str.values()))

        aw = jnp.exp(s_win - m) * wband
        num = jax.lax.dot_general(
            aw, vw, (((1,), (0,)), ((), ())),
            preferred_element_type=jnp.float32)
        den = jnp.sum(aw, axis=1, keepdims=True)

        nterms, dterms = [num], [den]
        for j, off in live:
            ej = jnp.exp(s_str[j] - m) * ws_ref[0, 0, j]
            if n0 >= off:
                vb = v_ref[0, pl.ds(n0 - off, _TILE), :]
            else:
                d = off - n0
                ej = jnp.where(rcol >= d, ej, 0.0)
                vb = jnp.concatenate(
                    [v_ref[0, 0:d, :], v_ref[0, 0:_TILE - d, :]], axis=0)
            nterms.append(ej * vb)
            dterms.append(ej)
        num = _tree_sum(nterms)
        den = _tree_sum(dterms)
        out_ref[0, pl.ds(n0, _TILE), :] = num / den


def kernel(q, k, v, pos_bias):
    B, H, N, HD = q.shape

    covered = [o for o in _OFFSETS if o <= _LB]
    r = jnp.arange(_TILE)[:, None]
    c = jnp.arange(_KWIN)[None, :]
    off_mat = (r + _LB - c)[None]
    eb = jnp.exp(pos_bias)
    wband = jnp.zeros((H, _TILE, _KWIN), jnp.float32)
    for i, off in enumerate(covered):
        wband = jnp.where(off_mat == off, eb[i][:, None, None], wband)
    wstr = eb[_NCOV:, :].T
    wstr = jnp.pad(wstr, ((0, 0), (0, 1)))[:, None, :]

    out = pl.pallas_call(
        _attn_body,
        grid=(H,),
        in_specs=[
            pl.BlockSpec((1, 1, 8), lambda h: (h, 0, 0),
                         memory_space=pltpu.SMEM),
            pl.BlockSpec((1, _TILE, _KWIN), lambda h: (h, 0, 0)),
            pl.BlockSpec((1, 1, N, HD), lambda h: (0, h, 0, 0)),
            pl.BlockSpec((1, 1, N, HD), lambda h: (0, h, 0, 0)),
            pl.BlockSpec((1, 1, N, HD), lambda h: (0, h, 0, 0)),
        ],
        out_specs=pl.BlockSpec((1, 1, N, HD), lambda h: (0, h, 0, 0)),
        out_shape=jax.ShapeDtypeStruct((B, H, N, HD), jnp.float32),
    )(wstr, wband, q, k, v)
    return out


# speedup vs baseline: 1.0976x; 1.0976x over previous
import jax
import jax.numpy as jnp
from jax.experimental import pallas as pl
from jax.experimental.pallas import tpu as pltpu

_OFFSETS = tuple(list(range(33)) + [48, 64, 96, 128, 192, 256, 384, 512, 768, 1024, 1536])
_LB = 128
_NCOV = 37
_COVERED = tuple(o for o in _OFFSETS if o <= _LB)
_STRIDED = tuple(o for o in _OFFSETS if o > _LB)
_NEG = -1e30
_TILE = 128
_KWIN = _TILE + _LB
_N = 2048
_SC = 0.125


def _tree_sum(terms):
    while len(terms) > 1:
        nxt = [a + b for a, b in zip(terms[::2], terms[1::2])]
        if len(terms) % 2:
            nxt.append(terms[-1])
        terms = nxt
    return terms[0]


def _tree_max(terms):
    while len(terms) > 1:
        nxt = [jnp.maximum(a, b) for a, b in zip(terms[::2], terms[1::2])]
        if len(terms) % 2:
            nxt.append(terms[-1])
        terms = nxt
    return terms[0]


def _attn_body(eb3_ref, q4_ref, k4_ref, v4_ref, out4_ref):
    eb_ref = eb3_ref.at[0]
    q_ref = q4_ref.at[0]
    k_ref = k4_ref.at[0]
    v_ref = v4_ref.at[0]
    out_ref = out4_ref.at[0]

    r2 = jax.lax.broadcasted_iota(jnp.int32, (_TILE, _KWIN), 0)
    c2 = jax.lax.broadcasted_iota(jnp.int32, (_TILE, _KWIN), 1)
    dmat = r2 + _LB - c2
    wband = jnp.zeros((_TILE, _KWIN), jnp.float32)
    for i, off in enumerate(_COVERED):
        wband = jnp.where(dmat == off, eb_ref[0, i], wband)
    wpos = wband > 0.0

    rcol = jax.lax.broadcasted_iota(jnp.int32, (_TILE, 1), 0)

    for t in range(_N // _TILE):
        n0 = t * _TILE
        qb = q_ref[0, pl.ds(n0, _TILE), :] * _SC
        if t == 0:
            kw = jnp.concatenate(
                [k_ref[0, 0:_TILE, :], k_ref[0, 0:_TILE, :]], axis=0)
            vw = jnp.concatenate(
                [v_ref[0, 0:_TILE, :], v_ref[0, 0:_TILE, :]], axis=0)
        else:
            kw = k_ref[0, pl.ds(n0 - _LB, _KWIN), :]
            vw = v_ref[0, pl.ds(n0 - _LB, _KWIN), :]
        s_win = jax.lax.dot_general(
            qb, kw, (((1,), (1,)), ((), ())),
            preferred_element_type=jnp.float32)
        if t == 0:
            valid = wpos & (c2 >= _LB)
        else:
            valid = wpos
        s_win = jnp.where(valid, s_win, _NEG)
        m_win = jnp.max(s_win, axis=1, keepdims=True)

        live = [(j, off) for j, off in enumerate(_STRIDED) if n0 + _TILE > off]
        s_str = {}
        for j, off in live:
            if n0 >= off:
                kb = k_ref[0, pl.ds(n0 - off, _TILE), :]
            else:
                d = off - n0
                kb = jnp.concatenate(
                    [k_ref[0, 0:d, :], k_ref[0, 0:_TILE - d, :]], axis=0)
            sj = jnp.sum(qb * kb, axis=1, keepdims=True)
            if n0 < off:
                sj = jnp.where(rcol >= off - n0, sj, _NEG)
            s_str[j] = sj
        m = _tree_max([m_win] + list(s_str.values()))

        aw = jnp.exp(s_win - m) * wband
        num = jax.lax.dot_general(
            aw, vw, (((1,), (0,)), ((), ())),
            preferred_element_type=jnp.float32)
        den = jnp.sum(aw, axis=1, keepdims=True)

        nterms, dterms = [num], [den]
        for j, off in live:
            ej = jnp.exp(s_str[j] - m) * eb_ref[0, _NCOV + j]
            if n0 >= off:
                vb = v_ref[0, pl.ds(n0 - off, _TILE), :]
            else:
                d = off - n0
                ej = jnp.where(rcol >= d, ej, 0.0)
                vb = jnp.concatenate(
                    [v_ref[0, 0:d, :], v_ref[0, 0:_TILE - d, :]], axis=0)
            nterms.append(ej * vb)
            dterms.append(ej)
        num = _tree_sum(nterms)
        den = _tree_sum(dterms)
        out_ref[0, pl.ds(n0, _TILE), :] = num / den


def kernel(q, k, v, pos_bias):
    B, H, N, HD = q.shape
    eb = jnp.pad(jnp.exp(pos_bias).T, ((0, 0), (0, 4)))[:, None, :]

    out = pl.pallas_call(
        _attn_body,
        grid=(H,),
        in_specs=[
            pl.BlockSpec((1, 1, 48), lambda h: (h, 0, 0),
                         memory_space=pltpu.SMEM),
            pl.BlockSpec((1, 1, N, HD), lambda h: (0, h, 0, 0)),
            pl.BlockSpec((1, 1, N, HD), lambda h: (0, h, 0, 0)),
            pl.BlockSpec((1, 1, N, HD), lambda h: (0, h, 0, 0)),
        ],
        out_specs=pl.BlockSpec((1, 1, N, HD), lambda h: (0, h, 0, 0)),
        out_shape=jax.ShapeDtypeStruct((B, H, N, HD), jnp.float32),
    )(eb, q, k, v)
    return out


# speedup vs baseline: 1.1000x; 1.0022x over previous
import jax
import jax.numpy as jnp
from jax.experimental import pallas as pl
from jax.experimental.pallas import tpu as pltpu

_OFFSETS = tuple(list(range(33)) + [48, 64, 96, 128, 192, 256, 384, 512, 768, 1024, 1536])
_LB = 128
_NCOV = 37
_COVERED = tuple(o for o in _OFFSETS if o <= _LB)
_STRIDED = tuple(o for o in _OFFSETS if o > _LB)
_NEG = -1e30
_TILE = 128
_KWIN = _TILE + _LB
_N = 2048
_SC = 0.125


def _tree_sum(terms):
    while len(terms) > 1:
        nxt = [a + b for a, b in zip(terms[::2], terms[1::2])]
        if len(terms) % 2:
            nxt.append(terms[-1])
        terms = nxt
    return terms[0]


def _tree_max(terms):
    while len(terms) > 1:
        nxt = [jnp.maximum(a, b) for a, b in zip(terms[::2], terms[1::2])]
        if len(terms) % 2:
            nxt.append(terms[-1])
        terms = nxt
    return terms[0]


def _attn_body(eb3_ref, q4_ref, k4_ref, v4_ref, out4_ref):
    eb_ref = eb3_ref.at[0]
    q_ref = q4_ref.at[0]
    k_ref = k4_ref.at[0]
    v_ref = v4_ref.at[0]
    out_ref = out4_ref.at[0]

    r2 = jax.lax.broadcasted_iota(jnp.int32, (_TILE, _KWIN), 0)
    c2 = jax.lax.broadcasted_iota(jnp.int32, (_TILE, _KWIN), 1)
    dmat = r2 + _LB - c2
    wband = jnp.zeros((_TILE, _KWIN), jnp.float32)
    for i, off in enumerate(_COVERED):
        wband = jnp.where(dmat == off, eb_ref[0, i], wband)
    wpos = wband > 0.0

    rcol = jax.lax.broadcasted_iota(jnp.int32, (_TILE, 1), 0)

    for t in range(_N // _TILE):
        n0 = t * _TILE
        qb = q_ref[0, pl.ds(n0, _TILE), :] * _SC
        if t == 0:
            kw = jnp.concatenate(
                [k_ref[0, 0:_TILE, :], k_ref[0, 0:_TILE, :]], axis=0)
            vw = jnp.concatenate(
                [v_ref[0, 0:_TILE, :], v_ref[0, 0:_TILE, :]], axis=0)
        else:
            kw = k_ref[0, pl.ds(n0 - _LB, _KWIN), :]
            vw = v_ref[0, pl.ds(n0 - _LB, _KWIN), :]
        s_win = jax.lax.dot_general(
            qb, kw, (((1,), (1,)), ((), ())),
            preferred_element_type=jnp.float32)
        if t == 0:
            valid = wpos & (c2 >= _LB)
        else:
            valid = wpos
        s_win = jnp.where(valid, s_win, _NEG)
        m_win = jnp.max(s_win, axis=1, keepdims=True)

        live = [(j, off) for j, off in enumerate(_STRIDED) if n0 + _TILE > off]
        s_str = {}
        for j, off in live:
            if n0 >= off:
                kb = k_ref[0, pl.ds(n0 - off, _TILE), :]
            else:
                d = off - n0
                kb = jnp.concatenate(
                    [k_ref[0, 0:d, :], k_ref[0, 0:_TILE - d, :]], axis=0)
            sj = jnp.sum(qb * kb, axis=1, keepdims=True)
            if n0 < off:
                sj = jnp.where(rcol >= off - n0, sj, _NEG)
            s_str[j] = sj
        m = _tree_max([m_win] + list(s_str.values()))

        aw = jnp.exp(s_win - m) * wband
        num = jax.lax.dot_general(
            aw, vw, (((1,), (0,)), ((), ())),
            preferred_element_type=jnp.float32)
        den = jnp.sum(aw, axis=1, keepdims=True)

        nterms, dterms = [num], [den]
        for j, off in live:
            ej = jnp.exp(s_str[j] - m) * eb_ref[0, _NCOV + j]
            if n0 >= off:
                vb = v_ref[0, pl.ds(n0 - off, _TILE), :]
            else:
                d = off - n0
                ej = jnp.where(rcol >= d, ej, 0.0)
                vb = jnp.concatenate(
                    [v_ref[0, 0:d, :], v_ref[0, 0:_TILE - d, :]], axis=0)
            nterms.append(ej * vb)
            dterms.append(ej)
        num = _tree_sum(nterms)
        den = _tree_sum(dterms)
        out_ref[0, pl.ds(n0, _TILE), :] = num / den


def kernel(q, k, v, pos_bias):
    B, H, N, HD = q.shape
    eb = jnp.pad(jnp.exp(pos_bias).T, ((0, 0), (0, 4)))[:, None, :]

    out = pl.pallas_call(
        _attn_body,
        grid=(H,),
        in_specs=[
            pl.BlockSpec((1, 1, 48), lambda h: (h, 0, 0),
                         memory_space=pltpu.SMEM),
            pl.BlockSpec((1, 1, N, HD), lambda h: (0, h, 0, 0)),
            pl.BlockSpec((1, 1, N, HD), lambda h: (0, h, 0, 0)),
            pl.BlockSpec((1, 1, N, HD), lambda h: (0, h, 0, 0)),
        ],
        out_specs=pl.BlockSpec((1, 1, N, HD), lambda h: (0, h, 0, 0)),
        out_shape=jax.ShapeDtypeStruct((B, H, N, HD), jnp.float32),
        compiler_params=pltpu.CompilerParams(
            dimension_semantics=("parallel",)),
    )(eb, q, k, v)
    return out
